# Initial kernel scaffold; baseline (speedup 1.0000x reference)
import jax
import jax.numpy as jnp
from jax.experimental import pallas as pl


def kernel(batch, cand_vecs, W):
    raise NotImplementedError("write your pallas kernel here")


# speedup vs baseline: 6.8591x; 6.8591x over previous
import functools

import jax
import jax.numpy as jnp
from jax import lax
from jax.experimental import pallas as pl
from jax.experimental.pallas import tpu as pltpu
from jax.experimental.pallas import tpu_sc as plsc

VOCAB = 1000
HID = 128
B = 4096
L = 50

NC = 2
NS = 16
NW = NC * NS
ROWS = 2 * B
R_PER_W = ROWS // NW
PAIRS_PER_CHUNK = 8
CHUNK_ROWS = 2 * PAIRS_PER_CHUNK
N_CHUNKS = R_PER_W // CHUNK_ROWS
NSL = HID // 16

_mesh = plsc.VectorSubcoreMesh(core_axis_name="c", subcore_axis_name="s")


@functools.partial(
    pl.kernel,
    out_type=jax.ShapeDtypeStruct((ROWS, HID), jnp.float32),
    mesh=_mesh,
    scratch_types=[
        pltpu.VMEM((PAIRS_PER_CHUNK, 2 * L), jnp.int32),
        pltpu.VMEM((PAIRS_PER_CHUNK * 2 * L, HID), jnp.float32),
        pltpu.VMEM((CHUNK_ROWS, HID), jnp.float32),
        pltpu.SemaphoreType.DMA,
    ],
)
def _encode_sc(idx_hbm, table_hbm, out_hbm, idx_v, rows_v, acc_v, sem):
    wid = lax.axis_index("s") * NC + lax.axis_index("c")
    base = wid * R_PER_W

    def chunk_body(c, carry):
        row0 = pl.multiple_of(base + c * CHUNK_ROWS, CHUNK_ROWS)
        pltpu.sync_copy(
            idx_hbm.at[pl.ds(pl.multiple_of(row0 // 2, 8), PAIRS_PER_CHUNK)],
            idx_v,
        )
        cps = [
            pltpu.async_copy(
                table_hbm.at[idx_v.at[j]],
                rows_v.at[pl.ds(j * 2 * L, 2 * L)],
                sem,
            )
            for j in range(PAIRS_PER_CHUNK)
        ]
        for cp in cps:
            cp.wait()
        for r in range(CHUNK_ROWS):
            b0 = (r // 2) * 2 * L + (r % 2) * L

            def add_row(l, accs):
                return tuple(
                    accs[s] + rows_v[b0 + l, pl.ds(s * 16, 16)]
                    for s in range(NSL)
                )

            init = tuple(rows_v[b0, pl.ds(s * 16, 16)] for s in range(NSL))
            accs = lax.fori_loop(1, L, add_row, init)
            for s in range(NSL):
                acc_v[r, pl.ds(s * 16, 16)] = accs[s]
        pltpu.sync_copy(acc_v, out_hbm.at[pl.ds(row0, CHUNK_ROWS)])
        return carry

    lax.fori_loop(0, N_CHUNKS, chunk_body, 0)


def _matmul_body(ctx_ref, cand_ref, o_ref):
    o_ref[...] = lax.dot_general(
        ctx_ref[...],
        cand_ref[...],
        (((1,), (1,)), ((), ())),
        preferred_element_type=jnp.float32,
    ) * (1.0 / (L * L))


def _similarity(ctx, cand):
    BM = 512
    BN = 512
    return pl.pallas_call(
        _matmul_body,
        grid=(B // BM, B // BN),
        in_specs=[
            pl.BlockSpec((BM, HID), lambda i, j: (i, 0)),
            pl.BlockSpec((BN, HID), lambda i, j: (j, 0)),
        ],
        out_specs=pl.BlockSpec((BM, BN), lambda i, j: (i, j)),
        out_shape=jax.ShapeDtypeStruct((B, B), jnp.float32),
    )(ctx, cand)


def kernel(batch, cand_vecs, W):
    idx_all = jnp.concatenate(
        [batch.astype(jnp.int32), cand_vecs.astype(jnp.int32)], axis=0
    ).reshape(ROWS // 2, 2 * L)
    encs = _encode_sc(idx_all, W)
    return _similarity(encs[:B], encs[B:])


# speedup vs baseline: 9.3391x; 1.3616x over previous
import functools

import jax
import jax.numpy as jnp
from jax import lax
from jax.experimental import pallas as pl
from jax.experimental.pallas import tpu as pltpu
from jax.experimental.pallas import tpu_sc as plsc

VOCAB = 1000
HID = 128
B = 4096
L = 50

NC = 2
NS = 16
NW = NC * NS
ROWS = 2 * B
R_PER_W = ROWS // NW
PAIRS_PER_CHUNK = 8
CHUNK_ROWS = 2 * PAIRS_PER_CHUNK
N_CHUNKS = R_PER_W // CHUNK_ROWS
HIDW = HID // 2
NQ = HIDW // 16

_mesh = plsc.VectorSubcoreMesh(core_axis_name="c", subcore_axis_name="s")


@functools.partial(
    pl.kernel,
    out_type=jax.ShapeDtypeStruct((ROWS, HID), jnp.float32),
    mesh=_mesh,
    scratch_types=[
        pltpu.VMEM((2, PAIRS_PER_CHUNK, 2 * L), jnp.int32),
        pltpu.VMEM((2, PAIRS_PER_CHUNK * 2 * L, HIDW), jnp.int32),
        pltpu.VMEM((CHUNK_ROWS, HID), jnp.float32),
        pltpu.SemaphoreType.DMA,
        pltpu.SemaphoreType.DMA,
    ],
    compiler_params=pltpu.CompilerParams(use_tc_tiling_on_sc=False),
)
def _encode_sc(idx_hbm, table_hbm, out_hbm, idx_v, rows_v, acc_v, sem0, sem1):
    wid = lax.axis_index("s") * NC + lax.axis_index("c")
    base = wid * R_PER_W
    sems = (sem0, sem1)

    def chunk_row0(c):
        return pl.multiple_of(base + c * CHUNK_ROWS, CHUNK_ROWS)

    def stage_and_fire(c, buf):
        row0 = chunk_row0(c)
        pltpu.sync_copy(
            idx_hbm.at[pl.ds(pl.multiple_of(row0 // 2, 8), PAIRS_PER_CHUNK)],
            idx_v.at[buf],
        )
        for j in range(PAIRS_PER_CHUNK):
            pltpu.async_copy(
                table_hbm.at[idx_v.at[buf].at[j]],
                rows_v.at[buf].at[pl.ds(j * 2 * L, 2 * L)],
                sems[buf],
            )

    def drain(c, buf):
        for j in range(PAIRS_PER_CHUNK):
            pltpu.make_async_copy(
                table_hbm.at[idx_v.at[buf].at[j]],
                rows_v.at[buf].at[pl.ds(j * 2 * L, 2 * L)],
                sems[buf],
            ).wait()

    def consume(c, buf):
        rows = rows_v.at[buf]
        mask = jnp.full((16,), -65536, jnp.int32)

        def row_body(r, carry):
            b0 = r * L
            lo = [jnp.zeros((16,), jnp.float32) for _ in range(NQ)]
            hi = [jnp.zeros((16,), jnp.float32) for _ in range(NQ)]
            for l in range(L):
                for s in range(NQ):
                    w = rows[b0 + l, pl.ds(s * 16, 16)]
                    lo[s] = lo[s] + lax.bitcast_convert_type(
                        lax.shift_left(w, 16), jnp.float32
                    )
                    hi[s] = hi[s] + lax.bitcast_convert_type(w & mask, jnp.float32)
            for s in range(NQ):
                acc_v[r, pl.ds(s * 16, 16)] = lo[s]
                acc_v[r, pl.ds(HIDW + s * 16, 16)] = hi[s]
            return carry

        lax.fori_loop(0, CHUNK_ROWS, row_body, 0)
        pltpu.sync_copy(acc_v, out_hbm.at[pl.ds(chunk_row0(c), CHUNK_ROWS)])

    stage_and_fire(0, 0)

    def ring_body(g, carry):
        for b in range(2):
            c = g * 2 + b
            nxt = c + 1

            @pl.when(nxt < N_CHUNKS)
            def _():
                stage_and_fire(nxt, (b + 1) % 2)

            drain(c, b)
            consume(c, b)
        return carry

    lax.fori_loop(0, N_CHUNKS // 2, ring_body, 0)


def _matmul_body(ctx_ref, cand_ref, o_ref):
    o_ref[...] = lax.dot_general(
        ctx_ref[...],
        cand_ref[...],
        (((1,), (1,)), ((), ())),
        preferred_element_type=jnp.float32,
    ) * (1.0 / (L * L))


def _similarity(ctx, cand):
    BM = 512
    BN = 512
    return pl.pallas_call(
        _matmul_body,
        grid=(B // BM, B // BN),
        in_specs=[
            pl.BlockSpec((BM, HID), lambda i, j: (i, 0)),
            pl.BlockSpec((BN, HID), lambda i, j: (j, 0)),
        ],
        out_specs=pl.BlockSpec((BM, BN), lambda i, j: (i, j)),
        out_shape=jax.ShapeDtypeStruct((B, B), jnp.float32),
    )(ctx, cand)


def kernel(batch, cand_vecs, W):
    idx_all = jnp.concatenate(
        [batch.astype(jnp.int32), cand_vecs.astype(jnp.int32)], axis=0
    ).reshape(ROWS // 2, 2 * L)
    Wb = W.astype(jnp.bfloat16)
    Wp = lax.bitcast_convert_type(
        jnp.stack([Wb[:, :HIDW], Wb[:, HIDW:]], axis=-1), jnp.int32
    )
    encs = _encode_sc(idx_all, Wp)
    return _similarity(encs[:B], encs[B:])


# speedup vs baseline: 9.8635x; 1.0562x over previous
import functools

import jax
import jax.numpy as jnp
from jax import lax
from jax.experimental import pallas as pl
from jax.experimental.pallas import tpu as pltpu
from jax.experimental.pallas import tpu_sc as plsc

VOCAB = 1000
HID = 128
B = 4096
L = 50

NC = 2
NS = 16
NW = NC * NS
ROWS = 2 * B
R_PER_W = ROWS // NW
PAIRS_PER_CHUNK = 8
CHUNK_ROWS = 2 * PAIRS_PER_CHUNK
N_CHUNKS = R_PER_W // CHUNK_ROWS
HIDW = HID // 2
NQ = HIDW // 16

_mesh = plsc.VectorSubcoreMesh(core_axis_name="c", subcore_axis_name="s")


@functools.partial(
    pl.kernel,
    out_type=(
        jax.ShapeDtypeStruct((B, HID), jnp.float32),
        jax.ShapeDtypeStruct((B, HID), jnp.float32),
    ),
    mesh=_mesh,
    scratch_types=[
        pltpu.VMEM((2, PAIRS_PER_CHUNK, 2 * L), jnp.int32),
        pltpu.VMEM((2, PAIRS_PER_CHUNK * 2 * L, HIDW), jnp.int32),
        pltpu.VMEM((CHUNK_ROWS, HID), jnp.float32),
        pltpu.SemaphoreType.DMA,
        pltpu.SemaphoreType.DMA,
    ],
    compiler_params=pltpu.CompilerParams(use_tc_tiling_on_sc=False),
)
def _encode_sc(
    idx_hbm, table_hbm, ctx_hbm, cand_hbm, idx_v, rows_v, acc_v, sem0, sem1
):
    wid = lax.axis_index("s") * NC + lax.axis_index("c")
    half = wid // (NW // 2)
    base = wid * R_PER_W
    base_local = (wid % (NW // 2)) * R_PER_W
    sems = (sem0, sem1)

    def chunk_row0(c):
        return pl.multiple_of(base + c * CHUNK_ROWS, CHUNK_ROWS)

    def stage_and_fire(c, buf):
        row0 = chunk_row0(c)
        pltpu.sync_copy(
            idx_hbm.at[pl.ds(pl.multiple_of(row0 // 2, 8), PAIRS_PER_CHUNK)],
            idx_v.at[buf],
        )
        for j in range(PAIRS_PER_CHUNK):
            pltpu.async_copy(
                table_hbm.at[idx_v.at[buf].at[j]],
                rows_v.at[buf].at[pl.ds(j * 2 * L, 2 * L)],
                sems[buf],
            )

    def drain(c, buf):
        for j in range(PAIRS_PER_CHUNK):
            pltpu.make_async_copy(
                table_hbm.at[idx_v.at[buf].at[j]],
                rows_v.at[buf].at[pl.ds(j * 2 * L, 2 * L)],
                sems[buf],
            ).wait()

    def consume(c, buf):
        rows = rows_v.at[buf]
        mask = jnp.full((16,), -65536, jnp.int32)

        def row_body(r, carry):
            b0 = r * L
            lo = [jnp.zeros((16,), jnp.float32) for _ in range(NQ)]
            hi = [jnp.zeros((16,), jnp.float32) for _ in range(NQ)]
            for l in range(L):
                for s in range(NQ):
                    w = rows[b0 + l, pl.ds(s * 16, 16)]
                    lo[s] = lo[s] + lax.bitcast_convert_type(
                        lax.shift_left(w, 16), jnp.float32
                    )
                    hi[s] = hi[s] + lax.bitcast_convert_type(w & mask, jnp.float32)
            for s in range(NQ):
                acc_v[r, pl.ds(s * 16, 16)] = lo[s]
                acc_v[r, pl.ds(HIDW + s * 16, 16)] = hi[s]
            return carry

        lax.fori_loop(0, CHUNK_ROWS, row_body, 0)
        dst = pl.ds(
            pl.multiple_of(base_local + c * CHUNK_ROWS, CHUNK_ROWS),
            CHUNK_ROWS,
        )

        @pl.when(half == 0)
        def _():
            pltpu.sync_copy(acc_v, ctx_hbm.at[dst])

        @pl.when(half == 1)
        def _():
            pltpu.sync_copy(acc_v, cand_hbm.at[dst])

    stage_and_fire(0, 0)

    def ring_body(g, carry):
        for b in range(2):
            c = g * 2 + b
            nxt = c + 1

            @pl.when(nxt < N_CHUNKS)
            def _():
                stage_and_fire(nxt, (b + 1) % 2)

            drain(c, b)
            consume(c, b)
        return carry

    lax.fori_loop(0, N_CHUNKS // 2, ring_body, 0)


def _matmul_body(ctx_ref, cand_ref, o_ref):
    o_ref[...] = lax.dot_general(
        ctx_ref[...].astype(jnp.bfloat16),
        cand_ref[...].astype(jnp.bfloat16),
        (((1,), (1,)), ((), ())),
        preferred_element_type=jnp.float32,
    ) * (1.0 / (L * L))


def _similarity(ctx, cand):
    BM = 512
    BN = 512
    return pl.pallas_call(
        _matmul_body,
        grid=(B // BM, B // BN),
        in_specs=[
            pl.BlockSpec((BM, HID), lambda i, j: (i, 0)),
            pl.BlockSpec((BN, HID), lambda i, j: (j, 0)),
        ],
        out_specs=pl.BlockSpec((BM, BN), lambda i, j: (i, j)),
        out_shape=jax.ShapeDtypeStruct((B, B), jnp.float32),
    )(ctx, cand)


def kernel(batch, cand_vecs, W):
    idx_all = jnp.concatenate(
        [batch.astype(jnp.int32), cand_vecs.astype(jnp.int32)], axis=0
    ).reshape(ROWS // 2, 2 * L)
    Wb = W.astype(jnp.bfloat16)
    Wp = lax.bitcast_convert_type(
        jnp.stack([Wb[:, :HIDW], Wb[:, HIDW:]], axis=-1), jnp.int32
    )
    ctx, cand = _encode_sc(idx_all, Wp)
    return _similarity(ctx, cand)


# speedup vs baseline: 11.9020x; 1.2067x over previous
import functools

import jax
import jax.numpy as jnp
from jax import lax
from jax.experimental import pallas as pl
from jax.experimental.pallas import tpu as pltpu
from jax.experimental.pallas import tpu_sc as plsc

VOCAB = 1000
HID = 128
B = 4096
L = 50

NC = 2
NS = 16
NW = NC * NS
ROWS = 2 * B
R_PER_W = ROWS // NW
PAIRS_PER_CHUNK = 8
CHUNK_ROWS = 2 * PAIRS_PER_CHUNK
N_CHUNKS = R_PER_W // CHUNK_ROWS
HIDW = HID // 2
NQ = HIDW // 16

_mesh = plsc.VectorSubcoreMesh(core_axis_name="c", subcore_axis_name="s")


@functools.partial(
    pl.kernel,
    out_type=(
        jax.ShapeDtypeStruct((B, HID), jnp.float32),
        jax.ShapeDtypeStruct((B, HID), jnp.float32),
    ),
    mesh=_mesh,
    scratch_types=[
        pltpu.VMEM((2, PAIRS_PER_CHUNK, 2 * L), jnp.int32),
        pltpu.VMEM((2, PAIRS_PER_CHUNK * 2 * L, HIDW), jnp.int32),
        pltpu.VMEM((CHUNK_ROWS, HID), jnp.float32),
        pltpu.SemaphoreType.DMA,
        pltpu.SemaphoreType.DMA,
    ],
    compiler_params=pltpu.CompilerParams(use_tc_tiling_on_sc=False),
)
def _encode_sc(
    idx_hbm, table_hbm, ctx_hbm, cand_hbm, idx_v, rows_v, acc_v, sem0, sem1
):
    wid = lax.axis_index("s") * NC + lax.axis_index("c")
    half = wid // (NW // 2)
    base = wid * R_PER_W
    base_local = (wid % (NW // 2)) * R_PER_W
    sems = (sem0, sem1)

    def chunk_row0(c):
        return pl.multiple_of(base + c * CHUNK_ROWS, CHUNK_ROWS)

    def stage_and_fire(c, buf):
        row0 = chunk_row0(c)
        pltpu.sync_copy(
            idx_hbm.at[pl.ds(pl.multiple_of(row0 // 2, 8), PAIRS_PER_CHUNK)],
            idx_v.at[buf],
        )
        for j in range(PAIRS_PER_CHUNK):
            pltpu.async_copy(
                table_hbm.at[idx_v.at[buf].at[j]],
                rows_v.at[buf].at[pl.ds(j * 2 * L, 2 * L)],
                sems[buf],
            )

    def drain(c, buf):
        for j in range(PAIRS_PER_CHUNK):
            pltpu.make_async_copy(
                table_hbm.at[idx_v.at[buf].at[j]],
                rows_v.at[buf].at[pl.ds(j * 2 * L, 2 * L)],
                sems[buf],
            ).wait()

    def consume(c, buf):
        rows = rows_v.at[buf]

        def row_body(r, carry):
            b0 = r * L
            lo = [jnp.zeros((16,), jnp.float32) for _ in range(NQ)]
            hi = [jnp.zeros((16,), jnp.float32) for _ in range(NQ)]
            for l in range(L):
                for s in range(NQ):
                    w = rows[b0 + l, pl.ds(s * 16, 16)]
                    lo[s] = lo[s] + lax.bitcast_convert_type(
                        lax.shift_left(w, 16), jnp.float32
                    )
                    hi[s] = hi[s] + lax.bitcast_convert_type(w, jnp.float32)
            for s in range(NQ):
                acc_v[r, pl.ds(s * 16, 16)] = lo[s]
                acc_v[r, pl.ds(HIDW + s * 16, 16)] = hi[s]
            return carry

        lax.fori_loop(0, CHUNK_ROWS, row_body, 0)
        dst = pl.ds(
            pl.multiple_of(base_local + c * CHUNK_ROWS, CHUNK_ROWS),
            CHUNK_ROWS,
        )

        @pl.when(half == 0)
        def _():
            pltpu.sync_copy(acc_v, ctx_hbm.at[dst])

        @pl.when(half == 1)
        def _():
            pltpu.sync_copy(acc_v, cand_hbm.at[dst])

    stage_and_fire(0, 0)

    def ring_body(g, carry):
        for b in range(2):
            c = g * 2 + b
            nxt = c + 1

            @pl.when(nxt < N_CHUNKS)
            def _():
                stage_and_fire(nxt, (b + 1) % 2)

            drain(c, b)
            consume(c, b)
        return carry

    lax.fori_loop(0, N_CHUNKS // 2, ring_body, 0)


def _matmul_body(ctx_ref, cand_ref, o_ref):
    o_ref[...] = lax.dot_general(
        ctx_ref[...].astype(jnp.bfloat16),
        cand_ref[...].astype(jnp.bfloat16),
        (((1,), (1,)), ((), ())),
        preferred_element_type=jnp.float32,
    ) * (1.0 / (L * L))


def _similarity(ctx, cand):
    BM = 1024
    BN = 1024
    return pl.pallas_call(
        _matmul_body,
        grid=(B // BM, B // BN),
        in_specs=[
            pl.BlockSpec((BM, HID), lambda i, j: (i, 0)),
            pl.BlockSpec((BN, HID), lambda i, j: (j, 0)),
        ],
        out_specs=pl.BlockSpec((BM, BN), lambda i, j: (i, j)),
        out_shape=jax.ShapeDtypeStruct((B, B), jnp.float32),
    )(ctx, cand)


def kernel(batch, cand_vecs, W):
    idx_all = jnp.concatenate(
        [batch.astype(jnp.int32), cand_vecs.astype(jnp.int32)], axis=0
    ).reshape(ROWS // 2, 2 * L)
    Wb = W.astype(jnp.bfloat16)
    Wp = lax.bitcast_convert_type(
        jnp.stack([Wb[:, :HIDW], Wb[:, HIDW:]], axis=-1), jnp.int32
    )
    ctx, cand = _encode_sc(idx_all, Wp)
    return _similarity(ctx, cand)


# speedup vs baseline: 12.3324x; 1.0362x over previous
import functools

import jax
import jax.numpy as jnp
from jax import lax
from jax.experimental import pallas as pl
from jax.experimental.pallas import tpu as pltpu
from jax.experimental.pallas import tpu_sc as plsc

VOCAB = 1000
HID = 128
B = 4096
L = 50

NC = 2
NS = 16
NW = NC * NS
ROWS = 2 * B
R_PER_W = ROWS // NW
CHUNK_ROWS = 16
N_CHUNKS = R_PER_W // CHUNK_ROWS
CHUNK_IDX = CHUNK_ROWS * L
HIDW = HID // 2
NQ = HIDW // 16
GATHER_SLICES = [(o, min(128, CHUNK_IDX - o)) for o in range(0, CHUNK_IDX, 128)]

_mesh = plsc.VectorSubcoreMesh(core_axis_name="c", subcore_axis_name="s")


@functools.partial(
    pl.kernel,
    out_type=(
        jax.ShapeDtypeStruct((B, HID), jnp.float32),
        jax.ShapeDtypeStruct((B, HID), jnp.float32),
    ),
    mesh=_mesh,
    scratch_types=[
        (pltpu.VMEM((CHUNK_IDX,), jnp.int32), pltpu.VMEM((CHUNK_IDX,), jnp.int32)),
        pltpu.VMEM((2, CHUNK_IDX, HIDW), jnp.int32),
        pltpu.VMEM((2, CHUNK_ROWS, HID), jnp.float32),
        (pltpu.SemaphoreType.DMA, pltpu.SemaphoreType.DMA),
        (pltpu.SemaphoreType.DMA, pltpu.SemaphoreType.DMA),
        (pltpu.SemaphoreType.DMA, pltpu.SemaphoreType.DMA),
    ],
    compiler_params=pltpu.CompilerParams(use_tc_tiling_on_sc=False),
)
def _encode_sc(
    bidx_hbm, cidx_hbm, table_hbm, ctx_hbm, cand_hbm,
    idx_v, rows_v, acc_v, gsem, isem, osem,
):
    wid = lax.axis_index("s") * NC + lax.axis_index("c")
    half = wid // (NW // 2)
    base_local = (wid % (NW // 2)) * R_PER_W

    def fire_idx(c, b):
        off = pl.multiple_of((base_local + c * CHUNK_ROWS) * L, CHUNK_IDX)
        src = pl.ds(off, CHUNK_IDX)

        @pl.when(half == 0)
        def _():
            pltpu.async_copy(bidx_hbm.at[src], idx_v[b], isem[b])

        @pl.when(half == 1)
        def _():
            pltpu.async_copy(cidx_hbm.at[src], idx_v[b], isem[b])

    def wait_idx(b):
        pltpu.make_async_copy(
            bidx_hbm.at[pl.ds(0, CHUNK_IDX)], idx_v[b], isem[b]
        ).wait()

    def fire_gathers(b):
        for o, n in GATHER_SLICES:
            pltpu.async_copy(
                table_hbm.at[idx_v[b].at[pl.ds(o, n)]],
                rows_v.at[b].at[pl.ds(o, n)],
                gsem[b],
            )

    def drain_gathers(b):
        for o, n in GATHER_SLICES:
            pltpu.make_async_copy(
                table_hbm.at[idx_v[b].at[pl.ds(o, n)]],
                rows_v.at[b].at[pl.ds(o, n)],
                gsem[b],
            ).wait()

    def fire_out(c, b):
        dst = pl.ds(
            pl.multiple_of(base_local + c * CHUNK_ROWS, CHUNK_ROWS),
            CHUNK_ROWS,
        )

        @pl.when(half == 0)
        def _():
            pltpu.async_copy(acc_v.at[b], ctx_hbm.at[dst], osem[b])

        @pl.when(half == 1)
        def _():
            pltpu.async_copy(acc_v.at[b], cand_hbm.at[dst], osem[b])

    def wait_out(b):
        pltpu.make_async_copy(
            acc_v.at[b], ctx_hbm.at[pl.ds(0, CHUNK_ROWS)], osem[b]
        ).wait()

    def consume(b):
        rows = rows_v.at[b]
        acc = acc_v.at[b]

        def row_body(r, carry):
            b0 = r * L
            lo = [jnp.zeros((16,), jnp.float32) for _ in range(NQ)]
            hi = [jnp.zeros((16,), jnp.float32) for _ in range(NQ)]
            for l in range(L):
                for s in range(NQ):
                    w = rows[b0 + l, pl.ds(s * 16, 16)]
                    lo[s] = lo[s] + lax.bitcast_convert_type(
                        lax.shift_left(w, 16), jnp.float32
                    )
                    hi[s] = hi[s] + lax.bitcast_convert_type(w, jnp.float32)
            for s in range(NQ):
                acc[r, pl.ds(s * 16, 16)] = lo[s]
                acc[r, pl.ds(HIDW + s * 16, 16)] = hi[s]
            return carry

        lax.fori_loop(0, CHUNK_ROWS, row_body, 0)

    fire_idx(0, 0)
    wait_idx(0)
    fire_gathers(0)
    fire_idx(1, 1)

    def ring_body(g, carry):
        for b in range(2):
            c = g * 2 + b
            drain_gathers(b)

            @pl.when(c + 2 < N_CHUNKS)
            def _():
                fire_idx(c + 2, b)

            @pl.when(c + 1 < N_CHUNKS)
            def _():
                wait_idx(1 - b)
                fire_gathers(1 - b)

            @pl.when(c >= 2)
            def _():
                wait_out(b)

            consume(b)
            fire_out(c, b)
        return carry

    lax.fori_loop(0, N_CHUNKS // 2, ring_body, 0)
    wait_out(0)
    wait_out(1)


def _matmul_body(ctx_ref, cand_ref, o_ref):
    o_ref[...] = lax.dot_general(
        ctx_ref[...].astype(jnp.bfloat16),
        cand_ref[...].astype(jnp.bfloat16),
        (((1,), (1,)), ((), ())),
        preferred_element_type=jnp.float32,
    )


def _similarity(ctx, cand):
    BM = 1024
    BN = 1024
    return pl.pallas_call(
        _matmul_body,
        grid=(B // BM, B // BN),
        in_specs=[
            pl.BlockSpec((BM, HID), lambda i, j: (i, 0)),
            pl.BlockSpec((BN, HID), lambda i, j: (j, 0)),
        ],
        out_specs=pl.BlockSpec((BM, BN), lambda i, j: (i, j)),
        out_shape=jax.ShapeDtypeStruct((B, B), jnp.float32),
    )(ctx, cand)


def kernel(batch, cand_vecs, W):
    bidx = batch.astype(jnp.int32).reshape(B * L)
    cidx = cand_vecs.astype(jnp.int32).reshape(B * L)
    Wb = (W * (1.0 / L)).astype(jnp.bfloat16)
    Wp = lax.bitcast_convert_type(
        jnp.stack([Wb[:, :HIDW], Wb[:, HIDW:]], axis=-1), jnp.int32
    )
    ctx, cand = _encode_sc(bidx, cidx, Wp)
    return _similarity(ctx, cand)
